# Initial kernel scaffold; baseline (speedup 1.0000x reference)
import jax
import jax.numpy as jnp
from jax.experimental import pallas as pl


def kernel(x, edge_index, W, b):
    raise NotImplementedError("write your pallas kernel here")


# speedup vs baseline: 8.2156x; 8.2156x over previous
import functools

import jax
import jax.numpy as jnp
from jax import lax
from jax.experimental import pallas as pl
from jax.experimental.pallas import tpu as pltpu
from jax.experimental.pallas import tpu_sc as plsc

N_NODES = 10000
N_EDGES = 320000
D = 128

NC, NS = 2, 16
NW = NC * NS
CW = 128
NCHUNKS = N_EDGES // CW
CPT = NCHUNKS // NW
EXTRA = NCHUNKS - CPT * NW
SR = 10112
STRIPE = SR // NS
NFULL = STRIPE // CW
TAIL = STRIPE - NFULL * CW
CNTW = 16

_mesh = plsc.VectorSubcoreMesh(
    core_axis_name="c", subcore_axis_name="s", num_cores=NC, num_subcores=NS)


def _chunk_loop(wid, do_chunk):
    cbase = wid * CPT

    def body(i, carry):
        do_chunk(cbase + i)
        return carry

    lax.fori_loop(0, CPT, body, 0)

    @pl.when(wid < EXTRA)
    def _():
        do_chunk(NW * CPT + wid)


@functools.partial(
    pl.kernel,
    out_type=jax.ShapeDtypeStruct((NC * SR, D), jnp.float32),
    mesh=_mesh,
    scratch_types=[
        pltpu.VMEM((CW,), jnp.int32),
        pltpu.VMEM((CW,), jnp.int32),
        pltpu.VMEM((CW, D), jnp.float32),
        pltpu.VMEM_SHARED((SR, D), jnp.float32),
        pltpu.SemaphoreType.DMA,
    ],
)
def _sc_segment_sum(x_hbm, edge_hbm, zrows_hbm, s_out,
                    src_idx, dst_idx, rows, s_sh, sem):
    c = lax.axis_index("c")
    s = lax.axis_index("s")
    wid = c * NS + s

    pltpu.sync_copy(zrows_hbm, rows)
    base = s * STRIPE
    for k in range(NFULL):
        pltpu.sync_copy(rows, s_sh.at[pl.ds(base + k * CW, CW)])
    if TAIL:
        pltpu.sync_copy(rows.at[pl.ds(0, TAIL)],
                        s_sh.at[pl.ds(base + NFULL * CW, TAIL)])
    plsc.subcore_barrier()

    def do_chunk(ch):
        off = pl.multiple_of(ch * CW, CW)
        pltpu.sync_copy(edge_hbm.at[pl.ds(off, CW)], src_idx)
        pltpu.sync_copy(edge_hbm.at[pl.ds(N_EDGES + off, CW)], dst_idx)
        pltpu.async_copy(x_hbm.at[src_idx], rows, sem).wait()
        pltpu.sync_copy(rows, s_sh.at[dst_idx], add=True)

    _chunk_loop(wid, do_chunk)
    plsc.subcore_barrier()

    def writeback(r0, n):
        pltpu.sync_copy(s_sh.at[pl.ds(r0, n)], rows.at[pl.ds(0, n)])
        pltpu.sync_copy(rows.at[pl.ds(0, n)], s_out.at[pl.ds(c * SR + r0, n)])

    for k in range(NFULL):
        writeback(base + k * CW, CW)
    if TAIL:
        writeback(base + NFULL * CW, TAIL)


@functools.partial(
    pl.kernel,
    out_type=jax.ShapeDtypeStruct((NC * SR, D), jnp.float32),
    mesh=_mesh,
    scratch_types=[
        pltpu.VMEM((CW,), jnp.int32),
        pltpu.VMEM((CW, D), jnp.float32),
        pltpu.VMEM((CW, D), jnp.float32),
        pltpu.VMEM_SHARED((SR, D), jnp.float32),
    ],
)
def _sc_degree(edge_hbm, orows_hbm, zrows_hbm, cnt_out,
               dst_idx, ones_b, tmp_cnt, cnt_sh):
    c = lax.axis_index("c")
    s = lax.axis_index("s")
    wid = c * NS + s

    pltpu.sync_copy(orows_hbm, ones_b)
    pltpu.sync_copy(zrows_hbm, tmp_cnt)
    base = s * STRIPE
    for k in range(NFULL):
        pltpu.sync_copy(tmp_cnt, cnt_sh.at[pl.ds(base + k * CW, CW)])
    if TAIL:
        pltpu.sync_copy(tmp_cnt.at[pl.ds(0, TAIL)],
                        cnt_sh.at[pl.ds(base + NFULL * CW, TAIL)])
    plsc.subcore_barrier()

    def do_chunk(ch):
        off = pl.multiple_of(ch * CW, CW)
        pltpu.sync_copy(edge_hbm.at[pl.ds(N_EDGES + off, CW)], dst_idx)
        pltpu.sync_copy(ones_b, cnt_sh.at[dst_idx], add=True)

    _chunk_loop(wid, do_chunk)
    plsc.subcore_barrier()

    def writeback(r0, n):
        pltpu.sync_copy(cnt_sh.at[pl.ds(r0, n)], tmp_cnt.at[pl.ds(0, n)])
        pltpu.sync_copy(tmp_cnt.at[pl.ds(0, n)], cnt_out.at[pl.ds(c * SR + r0, n)])

    for k in range(NFULL):
        writeback(base + k * CW, CW)
    if TAIL:
        writeback(base + NFULL * CW, TAIL)


RB = 1000


def _tc_body(x_ref, s_ref, c_ref, wi_ref, wj_ref, b_ref, o_ref):
    cnt = c_ref[0, :, 0:1] + c_ref[1, :, 0:1]
    ssum = s_ref[0] + s_ref[1]
    smean = ssum / jnp.maximum(cnt, 1.0)
    o = (jnp.dot(x_ref[...], wi_ref[...], preferred_element_type=jnp.float32)
         + jnp.dot(smean, wj_ref[...], preferred_element_type=jnp.float32)
         + b_ref[...])
    o_ref[...] = jnp.where(cnt > 0.0, o, 0.0)


def kernel(x, edge_index, W, b):
    edge1d = edge_index.astype(jnp.int32).reshape(2 * N_EDGES)
    zrows = jnp.zeros((CW, D), jnp.float32)
    orows = jnp.ones((CW, D), jnp.float32)

    s_out = _sc_segment_sum(x, edge1d, zrows)
    cnt_out = _sc_degree(edge1d, orows, zrows)
    s3 = s_out.reshape(NC, SR, D)
    c3 = cnt_out.reshape(NC, SR, D)

    wi_t = W[:, :D].T
    wj_t = W[:, D:].T
    b2 = b.reshape(1, D)

    out = pl.pallas_call(
        _tc_body,
        grid=(N_NODES // RB,),
        in_specs=[
            pl.BlockSpec((RB, D), lambda i: (i, 0)),
            pl.BlockSpec((NC, RB, D), lambda i: (0, i, 0)),
            pl.BlockSpec((NC, RB, D), lambda i: (0, i, 0)),
            pl.BlockSpec((D, D), lambda i: (0, 0)),
            pl.BlockSpec((D, D), lambda i: (0, 0)),
            pl.BlockSpec((1, D), lambda i: (0, 0)),
        ],
        out_specs=pl.BlockSpec((RB, D), lambda i: (i, 0)),
        out_shape=jax.ShapeDtypeStruct((N_NODES, D), jnp.float32),
    )(x, s3, c3, wi_t, wj_t, b2)
    return out


# speedup vs baseline: 10.3628x; 1.2614x over previous
import functools

import jax
import jax.numpy as jnp
from jax import lax
from jax.experimental import pallas as pl
from jax.experimental.pallas import tpu as pltpu
from jax.experimental.pallas import tpu_sc as plsc

N_NODES = 10000
N_EDGES = 320000
D = 128

NC, NS = 2, 16
NW = NC * NS
CW = 128
NCHUNKS = N_EDGES // CW
CPT = NCHUNKS // NW
EXTRA = NCHUNKS - CPT * NW
SR = 10112
STRIPE = SR // NS
NFULL = STRIPE // CW
TAIL = STRIPE - NFULL * CW

_mesh = plsc.VectorSubcoreMesh(
    core_axis_name="c", subcore_axis_name="s", num_cores=NC, num_subcores=NS)


@functools.partial(
    pl.kernel,
    out_type=jax.ShapeDtypeStruct((NC * SR, D), jnp.float32),
    mesh=_mesh,
    scratch_types=[
        pltpu.VMEM((CW,), jnp.int32),
        pltpu.VMEM((CW,), jnp.int32),
        pltpu.VMEM((CW,), jnp.int32),
        pltpu.VMEM((CW,), jnp.int32),
        pltpu.VMEM((CW, D), jnp.float32),
        pltpu.VMEM((CW, D), jnp.float32),
        pltpu.VMEM_SHARED((SR, D), jnp.float32),
        pltpu.SemaphoreType.DMA,
        pltpu.SemaphoreType.DMA,
        pltpu.SemaphoreType.DMA,
        pltpu.SemaphoreType.DMA,
    ],
)
def _sc_segment_sum(x_hbm, edge_hbm, zrows_hbm, s_out,
                    si0, di0, si1, di1, rows0, rows1, s_sh,
                    sem_g0, sem_g1, sem_s0, sem_s1):
    c = lax.axis_index("c")
    s = lax.axis_index("s")
    wid = c * NS + s
    si = (si0, si1)
    di = (di0, di1)
    rows = (rows0, rows1)
    sg = (sem_g0, sem_g1)
    ss = (sem_s0, sem_s1)

    pltpu.sync_copy(zrows_hbm, rows0)
    base = s * STRIPE
    for k in range(NFULL):
        pltpu.sync_copy(rows0, s_sh.at[pl.ds(base + k * CW, CW)])
    if TAIL:
        pltpu.sync_copy(rows0.at[pl.ds(0, TAIL)],
                        s_sh.at[pl.ds(base + NFULL * CW, TAIL)])
    plsc.subcore_barrier()

    cbase = wid * CPT

    def load_idx(b, ch):
        off = pl.multiple_of(ch * CW, CW)
        pltpu.sync_copy(edge_hbm.at[pl.ds(off, CW)], si[b])
        pltpu.sync_copy(edge_hbm.at[pl.ds(N_EDGES + off, CW)], di[b])

    def start_gather(b):
        pltpu.async_copy(x_hbm.at[si[b]], rows[b], sg[b])

    def wait_gather(b):
        pltpu.make_async_copy(x_hbm.at[si[b]], rows[b], sg[b]).wait()

    def start_scatter(b):
        pltpu.async_copy(rows[b], s_sh.at[di[b]], ss[b], add=True)

    def wait_scatter(b):
        pltpu.make_async_copy(rows[b], s_sh.at[di[b]], ss[b]).wait()

    load_idx(0, cbase)
    start_gather(0)
    wait_gather(0)
    start_scatter(0)
    load_idx(1, cbase + 1)
    start_gather(1)
    wait_gather(1)
    start_scatter(1)
    wait_scatter(0)
    load_idx(0, cbase + 2)
    start_gather(0)

    def pair(p, carry):
        c0 = cbase + 2 * p
        wait_gather(0)
        start_scatter(0)
        wait_scatter(1)
        load_idx(1, c0 + 1)
        start_gather(1)
        wait_gather(1)
        start_scatter(1)
        wait_scatter(0)
        load_idx(0, c0 + 2)
        start_gather(0)
        return carry

    lax.fori_loop(1, CPT // 2 - 1, pair, 0)

    wait_gather(0)
    start_scatter(0)
    wait_scatter(1)
    load_idx(1, cbase + CPT - 1)
    start_gather(1)
    wait_gather(1)
    start_scatter(1)
    wait_scatter(0)
    wait_scatter(1)

    @pl.when(wid < EXTRA)
    def _():
        load_idx(0, NW * CPT + wid)
        start_gather(0)
        wait_gather(0)
        start_scatter(0)
        wait_scatter(0)

    plsc.subcore_barrier()

    def writeback(r0, n):
        pltpu.sync_copy(s_sh.at[pl.ds(r0, n)], rows0.at[pl.ds(0, n)])
        pltpu.sync_copy(rows0.at[pl.ds(0, n)], s_out.at[pl.ds(c * SR + r0, n)])

    for k in range(NFULL):
        writeback(base + k * CW, CW)
    if TAIL:
        writeback(base + NFULL * CW, TAIL)


@functools.partial(
    pl.kernel,
    out_type=jax.ShapeDtypeStruct((NC * SR, D), jnp.float32),
    mesh=_mesh,
    scratch_types=[
        pltpu.VMEM((CW,), jnp.int32),
        pltpu.VMEM((CW,), jnp.int32),
        pltpu.VMEM((CW, D), jnp.float32),
        pltpu.VMEM((CW, D), jnp.float32),
        pltpu.VMEM_SHARED((SR, D), jnp.float32),
        pltpu.SemaphoreType.DMA,
        pltpu.SemaphoreType.DMA,
    ],
)
def _sc_degree(edge_hbm, orows_hbm, zrows_hbm, cnt_out,
               di0, di1, ones_b, tmp_cnt, cnt_sh, sem_s0, sem_s1):
    c = lax.axis_index("c")
    s = lax.axis_index("s")
    wid = c * NS + s
    di = (di0, di1)
    ss = (sem_s0, sem_s1)

    pltpu.sync_copy(orows_hbm, ones_b)
    pltpu.sync_copy(zrows_hbm, tmp_cnt)
    base = s * STRIPE
    for k in range(NFULL):
        pltpu.sync_copy(tmp_cnt, cnt_sh.at[pl.ds(base + k * CW, CW)])
    if TAIL:
        pltpu.sync_copy(tmp_cnt.at[pl.ds(0, TAIL)],
                        cnt_sh.at[pl.ds(base + NFULL * CW, TAIL)])
    plsc.subcore_barrier()

    cbase = wid * CPT

    def load_idx(b, ch):
        off = pl.multiple_of(ch * CW, CW)
        pltpu.sync_copy(edge_hbm.at[pl.ds(N_EDGES + off, CW)], di[b])

    def start_scatter(b):
        pltpu.async_copy(ones_b, cnt_sh.at[di[b]], ss[b], add=True)

    def wait_scatter(b):
        pltpu.make_async_copy(ones_b, cnt_sh.at[di[b]], ss[b]).wait()

    load_idx(0, cbase)
    start_scatter(0)
    load_idx(1, cbase + 1)
    start_scatter(1)

    def pair(p, carry):
        c0 = cbase + 2 * p
        wait_scatter(0)
        load_idx(0, c0)
        start_scatter(0)
        wait_scatter(1)
        load_idx(1, c0 + 1)
        start_scatter(1)
        return carry

    lax.fori_loop(1, CPT // 2, pair, 0)
    wait_scatter(0)
    wait_scatter(1)

    @pl.when(wid < EXTRA)
    def _():
        load_idx(0, NW * CPT + wid)
        start_scatter(0)
        wait_scatter(0)

    plsc.subcore_barrier()

    def writeback(r0, n):
        pltpu.sync_copy(cnt_sh.at[pl.ds(r0, n)], tmp_cnt.at[pl.ds(0, n)])
        pltpu.sync_copy(tmp_cnt.at[pl.ds(0, n)], cnt_out.at[pl.ds(c * SR + r0, n)])

    for k in range(NFULL):
        writeback(base + k * CW, CW)
    if TAIL:
        writeback(base + NFULL * CW, TAIL)


RB = 1000


def _tc_body(x_ref, s_ref, c_ref, wi_ref, wj_ref, b_ref, o_ref):
    cnt = c_ref[0, :, 0:1] + c_ref[1, :, 0:1]
    ssum = s_ref[0] + s_ref[1]
    smean = ssum / jnp.maximum(cnt, 1.0)
    o = (jnp.dot(x_ref[...], wi_ref[...], preferred_element_type=jnp.float32)
         + jnp.dot(smean, wj_ref[...], preferred_element_type=jnp.float32)
         + b_ref[...])
    o_ref[...] = jnp.where(cnt > 0.0, o, 0.0)


def kernel(x, edge_index, W, b):
    edge1d = edge_index.astype(jnp.int32).reshape(2 * N_EDGES)
    zrows = jnp.zeros((CW, D), jnp.float32)
    orows = jnp.ones((CW, D), jnp.float32)

    s_out = _sc_segment_sum(x, edge1d, zrows)
    cnt_out = _sc_degree(edge1d, orows, zrows)
    s3 = s_out.reshape(NC, SR, D)
    c3 = cnt_out.reshape(NC, SR, D)

    wi_t = W[:, :D].T
    wj_t = W[:, D:].T
    b2 = b.reshape(1, D)

    out = pl.pallas_call(
        _tc_body,
        grid=(N_NODES // RB,),
        in_specs=[
            pl.BlockSpec((RB, D), lambda i: (i, 0)),
            pl.BlockSpec((NC, RB, D), lambda i: (0, i, 0)),
            pl.BlockSpec((NC, RB, D), lambda i: (0, i, 0)),
            pl.BlockSpec((D, D), lambda i: (0, 0)),
            pl.BlockSpec((D, D), lambda i: (0, 0)),
            pl.BlockSpec((1, D), lambda i: (0, 0)),
        ],
        out_specs=pl.BlockSpec((RB, D), lambda i: (i, 0)),
        out_shape=jax.ShapeDtypeStruct((N_NODES, D), jnp.float32),
    )(x, s3, c3, wi_t, wj_t, b2)
    return out


# speedup vs baseline: 12.7647x; 1.2318x over previous
import functools

import jax
import jax.numpy as jnp
from jax import lax
from jax.experimental import pallas as pl
from jax.experimental.pallas import tpu as pltpu
from jax.experimental.pallas import tpu_sc as plsc

N_NODES = 10000
N_EDGES = 320000
D = 128

NC, NS = 2, 16
NW = NC * NS
CW = 128
NCHUNKS = N_EDGES // CW
CPT = NCHUNKS // NW
EXTRA = NCHUNKS - CPT * NW
SR = 10112
STRIPE = SR // NS
NFULL = STRIPE // CW
TAIL = STRIPE - NFULL * CW

_mesh = plsc.VectorSubcoreMesh(
    core_axis_name="c", subcore_axis_name="s", num_cores=NC, num_subcores=NS)


@functools.partial(
    pl.kernel,
    out_type=jax.ShapeDtypeStruct((NC * SR, D), jnp.float32),
    mesh=_mesh,
    scratch_types=[
        [pltpu.VMEM((CW,), jnp.int32) for _ in range(2)],
        [pltpu.VMEM((CW,), jnp.int32) for _ in range(2)],
        [pltpu.VMEM((CW, D), jnp.float32) for _ in range(2)],
        pltpu.VMEM_SHARED((SR, D), jnp.float32),
        [pltpu.SemaphoreType.DMA for _ in range(2)],
        [pltpu.SemaphoreType.DMA for _ in range(2)],
        [pltpu.SemaphoreType.DMA for _ in range(2)],
        [pltpu.SemaphoreType.DMA for _ in range(2)],
    ],
)
def _sc_segment_sum(x_hbm, edge_hbm, zrows_hbm, s_out,
                    si, di, rows, s_sh, sg, ss, sis, sid):
    c = lax.axis_index("c")
    s = lax.axis_index("s")
    wid = c * NS + s

    pltpu.sync_copy(zrows_hbm, rows[0])
    base = s * STRIPE
    for k in range(NFULL):
        pltpu.sync_copy(rows[0], s_sh.at[pl.ds(base + k * CW, CW)])
    if TAIL:
        pltpu.sync_copy(rows[0].at[pl.ds(0, TAIL)],
                        s_sh.at[pl.ds(base + NFULL * CW, TAIL)])
    plsc.subcore_barrier()

    cbase = wid * CPT

    def start_src(b, ch):
        off = pl.multiple_of(ch * CW, CW)
        pltpu.async_copy(edge_hbm.at[pl.ds(off, CW)], si[b], sis[b])

    def wait_src(b):
        pltpu.make_async_copy(edge_hbm.at[pl.ds(0, CW)], si[b], sis[b]).wait()

    def start_dst(b, ch):
        off = pl.multiple_of(ch * CW, CW)
        pltpu.async_copy(edge_hbm.at[pl.ds(N_EDGES + off, CW)], di[b], sid[b])

    def wait_dst(b):
        pltpu.make_async_copy(edge_hbm.at[pl.ds(0, CW)], di[b], sid[b]).wait()

    def start_gather(b):
        pltpu.async_copy(x_hbm.at[si[b]], rows[b], sg[b])

    def wait_gather(b):
        pltpu.make_async_copy(x_hbm.at[si[b]], rows[b], sg[b]).wait()

    def start_scatter(b):
        pltpu.async_copy(rows[b], s_sh.at[di[b]], ss[b], add=True)

    def wait_scatter(b):
        pltpu.make_async_copy(rows[b], s_sh.at[di[b]], ss[b]).wait()

    start_src(0, cbase)
    start_dst(0, cbase)
    start_src(1, cbase + 1)
    start_dst(1, cbase + 1)
    wait_src(0)
    start_gather(0)
    wait_gather(0)
    start_src(0, cbase + 2)
    wait_dst(0)
    start_scatter(0)
    wait_src(1)
    start_gather(1)
    wait_gather(1)
    start_src(1, cbase + 3)
    wait_dst(1)
    start_scatter(1)
    wait_scatter(0)
    start_dst(0, cbase + 2)
    wait_src(0)
    start_gather(0)

    def pair(p, carry):
        k = cbase + 2 * p
        wait_gather(0)
        start_src(0, k + 2)
        wait_dst(0)
        start_scatter(0)
        wait_scatter(1)
        start_dst(1, k + 1)
        wait_src(1)
        start_gather(1)
        wait_gather(1)
        start_src(1, k + 3)
        wait_dst(1)
        start_scatter(1)
        wait_scatter(0)
        start_dst(0, k + 2)
        wait_src(0)
        start_gather(0)
        return carry

    lax.fori_loop(1, CPT // 2 - 1, pair, 0)

    wait_gather(0)
    wait_dst(0)
    start_scatter(0)
    wait_scatter(1)
    start_dst(1, cbase + CPT - 1)
    wait_src(1)
    start_gather(1)
    wait_gather(1)
    wait_dst(1)
    start_scatter(1)
    wait_scatter(0)
    wait_scatter(1)

    @pl.when(wid < EXTRA)
    def _():
        ch = NW * CPT + wid
        start_src(0, ch)
        start_dst(0, ch)
        wait_src(0)
        wait_dst(0)
        start_gather(0)
        wait_gather(0)
        start_scatter(0)
        wait_scatter(0)

    plsc.subcore_barrier()

    def writeback(r0, n):
        pltpu.sync_copy(s_sh.at[pl.ds(r0, n)], rows[0].at[pl.ds(0, n)])
        pltpu.sync_copy(rows[0].at[pl.ds(0, n)], s_out.at[pl.ds(c * SR + r0, n)])

    for k in range(NFULL):
        writeback(base + k * CW, CW)
    if TAIL:
        writeback(base + NFULL * CW, TAIL)


@functools.partial(
    pl.kernel,
    out_type=jax.ShapeDtypeStruct((NC * SR, D), jnp.float32),
    mesh=_mesh,
    scratch_types=[
        [pltpu.VMEM((CW,), jnp.int32) for _ in range(2)],
        pltpu.VMEM((CW, D), jnp.float32),
        pltpu.VMEM((CW, D), jnp.float32),
        pltpu.VMEM_SHARED((SR, D), jnp.float32),
        [pltpu.SemaphoreType.DMA for _ in range(2)],
        [pltpu.SemaphoreType.DMA for _ in range(2)],
    ],
)
def _sc_degree(edge_hbm, orows_hbm, zrows_hbm, cnt_out,
               di, ones_b, tmp_cnt, cnt_sh, ss, sid):
    c = lax.axis_index("c")
    s = lax.axis_index("s")
    wid = c * NS + s

    pltpu.sync_copy(orows_hbm, ones_b)
    pltpu.sync_copy(zrows_hbm, tmp_cnt)
    base = s * STRIPE
    for k in range(NFULL):
        pltpu.sync_copy(tmp_cnt, cnt_sh.at[pl.ds(base + k * CW, CW)])
    if TAIL:
        pltpu.sync_copy(tmp_cnt.at[pl.ds(0, TAIL)],
                        cnt_sh.at[pl.ds(base + NFULL * CW, TAIL)])
    plsc.subcore_barrier()

    cbase = wid * CPT

    def start_dst(b, ch):
        off = pl.multiple_of(ch * CW, CW)
        pltpu.async_copy(edge_hbm.at[pl.ds(N_EDGES + off, CW)], di[b], sid[b])

    def wait_dst(b):
        pltpu.make_async_copy(edge_hbm.at[pl.ds(0, CW)], di[b], sid[b]).wait()

    def start_scatter(b):
        pltpu.async_copy(ones_b, cnt_sh.at[di[b]], ss[b], add=True)

    def wait_scatter(b):
        pltpu.make_async_copy(ones_b, cnt_sh.at[di[b]], ss[b]).wait()

    start_dst(0, cbase)
    start_dst(1, cbase + 1)
    wait_dst(0)
    start_scatter(0)
    wait_dst(1)
    start_scatter(1)

    def pair(p, carry):
        k = cbase + 2 * p
        wait_scatter(0)
        start_dst(0, k)
        wait_scatter(1)
        start_dst(1, k + 1)
        wait_dst(0)
        start_scatter(0)
        wait_dst(1)
        start_scatter(1)
        return carry

    lax.fori_loop(1, CPT // 2, pair, 0)
    wait_scatter(0)
    wait_scatter(1)

    @pl.when(wid < EXTRA)
    def _():
        start_dst(0, NW * CPT + wid)
        wait_dst(0)
        start_scatter(0)
        wait_scatter(0)

    plsc.subcore_barrier()

    def writeback(r0, n):
        pltpu.sync_copy(cnt_sh.at[pl.ds(r0, n)], tmp_cnt.at[pl.ds(0, n)])
        pltpu.sync_copy(tmp_cnt.at[pl.ds(0, n)], cnt_out.at[pl.ds(c * SR + r0, n)])

    for k in range(NFULL):
        writeback(base + k * CW, CW)
    if TAIL:
        writeback(base + NFULL * CW, TAIL)


RB = 1000


def _tc_body(x_ref, s_ref, c_ref, wi_ref, wj_ref, b_ref, o_ref):
    cnt = c_ref[0, :, 0:1] + c_ref[1, :, 0:1]
    ssum = s_ref[0] + s_ref[1]
    smean = ssum / jnp.maximum(cnt, 1.0)
    o = (jnp.dot(x_ref[...], wi_ref[...], preferred_element_type=jnp.float32)
         + jnp.dot(smean, wj_ref[...], preferred_element_type=jnp.float32)
         + b_ref[...])
    o_ref[...] = jnp.where(cnt > 0.0, o, 0.0)


def kernel(x, edge_index, W, b):
    edge1d = edge_index.astype(jnp.int32).reshape(2 * N_EDGES)
    zrows = jnp.zeros((CW, D), jnp.float32)
    orows = jnp.ones((CW, D), jnp.float32)

    s_out = _sc_segment_sum(x, edge1d, zrows)
    cnt_out = _sc_degree(edge1d, orows, zrows)
    s3 = s_out.reshape(NC, SR, D)
    c3 = cnt_out.reshape(NC, SR, D)

    wi_t = W[:, :D].T
    wj_t = W[:, D:].T
    b2 = b.reshape(1, D)

    out = pl.pallas_call(
        _tc_body,
        grid=(N_NODES // RB,),
        in_specs=[
            pl.BlockSpec((RB, D), lambda i: (i, 0)),
            pl.BlockSpec((NC, RB, D), lambda i: (0, i, 0)),
            pl.BlockSpec((NC, RB, D), lambda i: (0, i, 0)),
            pl.BlockSpec((D, D), lambda i: (0, 0)),
            pl.BlockSpec((D, D), lambda i: (0, 0)),
            pl.BlockSpec((1, D), lambda i: (0, 0)),
        ],
        out_specs=pl.BlockSpec((RB, D), lambda i: (i, 0)),
        out_shape=jax.ShapeDtypeStruct((N_NODES, D), jnp.float32),
    )(x, s3, c3, wi_t, wj_t, b2)
    return out


# speedup vs baseline: 13.2297x; 1.0364x over previous
import functools

import jax
import jax.numpy as jnp
from jax import lax
from jax.experimental import pallas as pl
from jax.experimental.pallas import tpu as pltpu
from jax.experimental.pallas import tpu_sc as plsc

N_NODES = 10000
N_EDGES = 320000
D = 128

NC, NS = 2, 16
NW = NC * NS
SR = 10112
STRIPE = SR // NS
BW = 128
NFULL = STRIPE // BW
TAIL = STRIPE - NFULL * BW

CW = 64
NCHUNKS = N_EDGES // CW
CPT = NCHUNKS // NW
EXTRA = NCHUNKS - CPT * NW
NB = 4

CWD = 128
NCHD = N_EDGES // CWD
CPTD = NCHD // NW
EXTRAD = NCHD - CPTD * NW
NBD = 8

_mesh = plsc.VectorSubcoreMesh(
    core_axis_name="c", subcore_axis_name="s", num_cores=NC, num_subcores=NS)


@functools.partial(
    pl.kernel,
    out_type=jax.ShapeDtypeStruct((NC * SR, D), jnp.float32),
    mesh=_mesh,
    scratch_types=[
        [pltpu.VMEM((CW,), jnp.int32) for _ in range(NB)],
        [pltpu.VMEM((CW,), jnp.int32) for _ in range(NB)],
        [pltpu.VMEM((CW, D), jnp.float32) for _ in range(NB)],
        pltpu.VMEM_SHARED((SR, D), jnp.float32),
        [pltpu.SemaphoreType.DMA for _ in range(NB)],
        [pltpu.SemaphoreType.DMA for _ in range(NB)],
        [pltpu.SemaphoreType.DMA for _ in range(NB)],
        [pltpu.SemaphoreType.DMA for _ in range(NB)],
    ],
)
def _sc_segment_sum(x_hbm, edge_hbm, zrows_hbm, s_out,
                    si, di, rows, s_sh, sg, ss, sis, sid):
    c = lax.axis_index("c")
    s = lax.axis_index("s")
    wid = c * NS + s

    pltpu.sync_copy(zrows_hbm.at[pl.ds(0, CW)], rows[0])
    pltpu.sync_copy(zrows_hbm.at[pl.ds(CW, CW)], rows[1])
    base = s * STRIPE

    def zero_block(r0, n):
        pltpu.sync_copy(rows[0].at[pl.ds(0, min(n, CW))],
                        s_sh.at[pl.ds(r0, min(n, CW))])
        if n > CW:
            pltpu.sync_copy(rows[1].at[pl.ds(0, n - CW)],
                            s_sh.at[pl.ds(r0 + CW, n - CW)])

    for k in range(NFULL):
        zero_block(base + k * BW, BW)
    if TAIL:
        zero_block(base + NFULL * BW, TAIL)
    plsc.subcore_barrier()

    cbase = wid * CPT

    def start_src(b, ch):
        off = pl.multiple_of(ch * CW, CW)
        pltpu.async_copy(edge_hbm.at[pl.ds(off, CW)], si[b], sis[b])

    def wait_src(b):
        pltpu.make_async_copy(edge_hbm.at[pl.ds(0, CW)], si[b], sis[b]).wait()

    def start_dst(b, ch):
        off = pl.multiple_of(ch * CW, CW)
        pltpu.async_copy(edge_hbm.at[pl.ds(N_EDGES + off, CW)], di[b], sid[b])

    def wait_dst(b):
        pltpu.make_async_copy(edge_hbm.at[pl.ds(0, CW)], di[b], sid[b]).wait()

    def start_gather(b):
        pltpu.async_copy(x_hbm.at[si[b]], rows[b], sg[b])

    def wait_gather(b):
        pltpu.make_async_copy(x_hbm.at[si[b]], rows[b], sg[b]).wait()

    def start_scatter(b):
        pltpu.async_copy(rows[b], s_sh.at[di[b]], ss[b], add=True)

    def wait_scatter(b):
        pltpu.make_async_copy(rows[b], s_sh.at[di[b]], ss[b]).wait()

    for b in range(NB):
        start_src(b, cbase + b)
        start_dst(b, cbase + b)
    for b in range(NB):
        wait_src(b)
        start_gather(b)
    for b in range(NB):
        wait_gather(b)
        wait_dst(b)
        start_scatter(b)

    def group(g, carry):
        l = cbase + g * NB
        for b in range(NB):
            wait_scatter(b)
            start_src(b, l + b)
            start_dst(b, l + b)
            wait_src(b)
            start_gather(b)
        for b in range(NB):
            wait_gather(b)
            wait_dst(b)
            start_scatter(b)
        return carry

    lax.fori_loop(1, CPT // NB, group, 0)
    for b in range(NB):
        wait_scatter(b)

    @pl.when(wid < EXTRA)
    def _():
        ch = NW * CPT + wid
        start_src(0, ch)
        start_dst(0, ch)
        wait_src(0)
        wait_dst(0)
        start_gather(0)
        wait_gather(0)
        start_scatter(0)
        wait_scatter(0)

    plsc.subcore_barrier()

    def writeback(r0, n):
        n0 = min(n, CW)
        pltpu.sync_copy(s_sh.at[pl.ds(r0, n0)], rows[0].at[pl.ds(0, n0)])
        pltpu.sync_copy(rows[0].at[pl.ds(0, n0)],
                        s_out.at[pl.ds(c * SR + r0, n0)])
        if n > CW:
            pltpu.sync_copy(s_sh.at[pl.ds(r0 + CW, n - CW)],
                            rows[1].at[pl.ds(0, n - CW)])
            pltpu.sync_copy(rows[1].at[pl.ds(0, n - CW)],
                            s_out.at[pl.ds(c * SR + r0 + CW, n - CW)])

    for k in range(NFULL):
        writeback(base + k * BW, BW)
    if TAIL:
        writeback(base + NFULL * BW, TAIL)


@functools.partial(
    pl.kernel,
    out_type=jax.ShapeDtypeStruct((NC * SR, D), jnp.float32),
    mesh=_mesh,
    scratch_types=[
        [pltpu.VMEM((CWD,), jnp.int32) for _ in range(NBD)],
        pltpu.VMEM((CWD, D), jnp.float32),
        pltpu.VMEM_SHARED((SR, D), jnp.float32),
        [pltpu.SemaphoreType.DMA for _ in range(NBD)],
        [pltpu.SemaphoreType.DMA for _ in range(NBD)],
    ],
)
def _sc_degree(edge_hbm, orows_hbm, zrows_hbm, cnt_out,
               di, buf, cnt_sh, ss, sid):
    c = lax.axis_index("c")
    s = lax.axis_index("s")
    wid = c * NS + s

    pltpu.sync_copy(zrows_hbm, buf)
    base = s * STRIPE
    for k in range(NFULL):
        pltpu.sync_copy(buf, cnt_sh.at[pl.ds(base + k * BW, BW)])
    if TAIL:
        pltpu.sync_copy(buf.at[pl.ds(0, TAIL)],
                        cnt_sh.at[pl.ds(base + NFULL * BW, TAIL)])
    pltpu.sync_copy(orows_hbm, buf)
    plsc.subcore_barrier()

    cbase = wid * CPTD

    def start_dst(b, ch):
        off = pl.multiple_of(ch * CWD, CWD)
        pltpu.async_copy(edge_hbm.at[pl.ds(N_EDGES + off, CWD)], di[b], sid[b])

    def wait_dst(b):
        pltpu.make_async_copy(edge_hbm.at[pl.ds(0, CWD)], di[b], sid[b]).wait()

    def start_scatter(b):
        pltpu.async_copy(buf, cnt_sh.at[di[b]], ss[b], add=True)

    def wait_scatter(b):
        pltpu.make_async_copy(buf, cnt_sh.at[di[b]], ss[b]).wait()

    for b in range(NBD):
        start_dst(b, cbase + b)
    for b in range(NBD):
        wait_dst(b)
        start_scatter(b)

    def group(g, carry):
        l = cbase + g * NBD
        for b in range(NBD):
            wait_scatter(b)
            start_dst(b, l + b)
        for b in range(NBD):
            wait_dst(b)
            start_scatter(b)
        return carry

    lax.fori_loop(1, CPTD // NBD, group, 0)
    REM = CPTD - (CPTD // NBD) * NBD
    for b in range(REM):
        wait_scatter(b)
        start_dst(b, cbase + (CPTD // NBD) * NBD + b)
    for b in range(REM):
        wait_dst(b)
        start_scatter(b)
    for b in range(NBD):
        wait_scatter(b)

    @pl.when(wid < EXTRAD)
    def _():
        start_dst(0, NW * CPTD + wid)
        wait_dst(0)
        start_scatter(0)
        wait_scatter(0)

    plsc.subcore_barrier()

    def writeback(r0, n):
        pltpu.sync_copy(cnt_sh.at[pl.ds(r0, n)], buf.at[pl.ds(0, n)])
        pltpu.sync_copy(buf.at[pl.ds(0, n)], cnt_out.at[pl.ds(c * SR + r0, n)])

    for k in range(NFULL):
        writeback(base + k * BW, BW)
    if TAIL:
        writeback(base + NFULL * BW, TAIL)


RB = 1000


def _tc_body(x_ref, s_ref, c_ref, wi_ref, wj_ref, b_ref, o_ref):
    cnt = c_ref[0, :, 0:1] + c_ref[1, :, 0:1]
    ssum = s_ref[0] + s_ref[1]
    smean = ssum / jnp.maximum(cnt, 1.0)
    o = (jnp.dot(x_ref[...], wi_ref[...], preferred_element_type=jnp.float32)
         + jnp.dot(smean, wj_ref[...], preferred_element_type=jnp.float32)
         + b_ref[...])
    o_ref[...] = jnp.where(cnt > 0.0, o, 0.0)


def kernel(x, edge_index, W, b):
    edge1d = edge_index.astype(jnp.int32).reshape(2 * N_EDGES)
    zrows = jnp.zeros((BW, D), jnp.float32)
    orows = jnp.ones((BW, D), jnp.float32)

    s_out = _sc_segment_sum(x, edge1d, zrows)
    cnt_out = _sc_degree(edge1d, orows, zrows)
    s3 = s_out.reshape(NC, SR, D)
    c3 = cnt_out.reshape(NC, SR, D)

    wi_t = W[:, :D].T
    wj_t = W[:, D:].T
    b2 = b.reshape(1, D)

    out = pl.pallas_call(
        _tc_body,
        grid=(N_NODES // RB,),
        in_specs=[
            pl.BlockSpec((RB, D), lambda i: (i, 0)),
            pl.BlockSpec((NC, RB, D), lambda i: (0, i, 0)),
            pl.BlockSpec((NC, RB, D), lambda i: (0, i, 0)),
            pl.BlockSpec((D, D), lambda i: (0, 0)),
            pl.BlockSpec((D, D), lambda i: (0, 0)),
            pl.BlockSpec((1, D), lambda i: (0, 0)),
        ],
        out_specs=pl.BlockSpec((RB, D), lambda i: (i, 0)),
        out_shape=jax.ShapeDtypeStruct((N_NODES, D), jnp.float32),
    )(x, s3, c3, wi_t, wj_t, b2)
    return out


# speedup vs baseline: 13.4982x; 1.0203x over previous
import functools

import jax
import jax.numpy as jnp
from jax import lax
from jax.experimental import pallas as pl
from jax.experimental.pallas import tpu as pltpu
from jax.experimental.pallas import tpu_sc as plsc

N_NODES = 10000
N_EDGES = 320000
D = 128

NC, NS = 2, 16
NW = NC * NS
SR = 10112
STRIPE = SR // NS
BW = 128
NFULL = STRIPE // BW
TAIL = STRIPE - NFULL * BW

CW = 64
NCHUNKS = N_EDGES // CW
CPT = NCHUNKS // NW
EXTRA = NCHUNKS - CPT * NW
NB = 4

CWD = 128
NCHD = N_EDGES // CWD
CPTD = NCHD // NW
EXTRAD = NCHD - CPTD * NW
NBD = 8

_mesh = plsc.VectorSubcoreMesh(
    core_axis_name="c", subcore_axis_name="s", num_cores=NC, num_subcores=NS)


@functools.partial(
    pl.kernel,
    out_type=[
        jax.ShapeDtypeStruct((NC * SR, D), jnp.float32),
        jax.ShapeDtypeStruct((NC * SR, D), jnp.float32),
    ],
    mesh=_mesh,
    scratch_types=[
        [pltpu.VMEM((CW,), jnp.int32) for _ in range(NB)],
        [pltpu.VMEM((CW,), jnp.int32) for _ in range(NB)],
        [pltpu.VMEM((CW, D), jnp.float32) for _ in range(NB)],
        pltpu.VMEM_SHARED((SR, D), jnp.float32),
        [pltpu.SemaphoreType.DMA for _ in range(NB)],
        [pltpu.SemaphoreType.DMA for _ in range(NB)],
        [pltpu.SemaphoreType.DMA for _ in range(NB)],
        [pltpu.SemaphoreType.DMA for _ in range(NB)],
    ],
)
def _sc_fused(x_hbm, edge_hbm, zrows_hbm, orows_hbm, s_out, cnt_out,
              si, di, rows, s_sh, sg, ss, sis, sid):
    c = lax.axis_index("c")
    s = lax.axis_index("s")
    wid = c * NS + s

    pltpu.sync_copy(zrows_hbm.at[pl.ds(0, CW)], rows[0])
    pltpu.sync_copy(zrows_hbm.at[pl.ds(CW, CW)], rows[1])
    base = s * STRIPE

    def zero_block(r0, n):
        pltpu.sync_copy(rows[0].at[pl.ds(0, min(n, CW))],
                        s_sh.at[pl.ds(r0, min(n, CW))])
        if n > CW:
            pltpu.sync_copy(rows[1].at[pl.ds(0, n - CW)],
                            s_sh.at[pl.ds(r0 + CW, n - CW)])

    for k in range(NFULL):
        zero_block(base + k * BW, BW)
    if TAIL:
        zero_block(base + NFULL * BW, TAIL)
    plsc.subcore_barrier()

    cbase = wid * CPT

    def start_src(b, ch):
        off = pl.multiple_of(ch * CW, CW)
        pltpu.async_copy(edge_hbm.at[pl.ds(off, CW)], si[b], sis[b])

    def wait_src(b):
        pltpu.make_async_copy(edge_hbm.at[pl.ds(0, CW)], si[b], sis[b]).wait()

    def start_dst(b, ch):
        off = pl.multiple_of(ch * CW, CW)
        pltpu.async_copy(edge_hbm.at[pl.ds(N_EDGES + off, CW)], di[b], sid[b])

    def wait_dst(b):
        pltpu.make_async_copy(edge_hbm.at[pl.ds(0, CW)], di[b], sid[b]).wait()

    def start_gather(b):
        pltpu.async_copy(x_hbm.at[si[b]], rows[b], sg[b])

    def wait_gather(b):
        pltpu.make_async_copy(x_hbm.at[si[b]], rows[b], sg[b]).wait()

    def start_scatter(b):
        pltpu.async_copy(rows[b], s_sh.at[di[b]], ss[b], add=True)

    def wait_scatter(b):
        pltpu.make_async_copy(rows[b], s_sh.at[di[b]], ss[b]).wait()

    for b in range(NB):
        start_src(b, cbase + b)
        start_dst(b, cbase + b)
    for b in range(NB):
        wait_src(b)
        start_gather(b)
    for b in range(NB):
        wait_gather(b)
        wait_dst(b)
        start_scatter(b)

    def group(g, carry):
        l = cbase + g * NB
        for b in range(NB):
            wait_scatter(b)
            start_src(b, l + b)
            start_dst(b, l + b)
            wait_src(b)
            start_gather(b)
        for b in range(NB):
            wait_gather(b)
            wait_dst(b)
            start_scatter(b)
        return carry

    lax.fori_loop(1, CPT // NB, group, 0)
    for b in range(NB):
        wait_scatter(b)

    @pl.when(wid < EXTRA)
    def _():
        ch = NW * CPT + wid
        start_src(0, ch)
        start_dst(0, ch)
        wait_src(0)
        wait_dst(0)
        start_gather(0)
        wait_gather(0)
        start_scatter(0)
        wait_scatter(0)

    plsc.subcore_barrier()

    def writeback(r0, n):
        n0 = min(n, CW)
        pltpu.sync_copy(s_sh.at[pl.ds(r0, n0)], rows[0].at[pl.ds(0, n0)])
        pltpu.sync_copy(rows[0].at[pl.ds(0, n0)],
                        s_out.at[pl.ds(c * SR + r0, n0)])
        if n > CW:
            pltpu.sync_copy(s_sh.at[pl.ds(r0 + CW, n - CW)],
                            rows[1].at[pl.ds(0, n - CW)])
            pltpu.sync_copy(rows[1].at[pl.ds(0, n - CW)],
                            s_out.at[pl.ds(c * SR + r0 + CW, n - CW)])

    for k in range(NFULL):
        writeback(base + k * BW, BW)
    if TAIL:
        writeback(base + NFULL * BW, TAIL)

    pltpu.sync_copy(zrows_hbm.at[pl.ds(0, CW)], rows[0])
    pltpu.sync_copy(zrows_hbm.at[pl.ds(CW, CW)], rows[1])
    for k in range(NFULL):
        zero_block(base + k * BW, BW)
    if TAIL:
        zero_block(base + NFULL * BW, TAIL)
    pltpu.sync_copy(orows_hbm.at[pl.ds(0, CW)], rows[0])
    plsc.subcore_barrier()

    def start_cscatter(b):
        pltpu.async_copy(rows[0], s_sh.at[di[b]], ss[b], add=True)

    def wait_cscatter(b):
        pltpu.make_async_copy(rows[0], s_sh.at[di[b]], ss[b]).wait()

    for b in range(NB):
        start_dst(b, cbase + b)
    for b in range(NB):
        wait_dst(b)
        start_cscatter(b)

    def cgroup(g, carry):
        l = cbase + g * NB
        for b in range(NB):
            wait_cscatter(b)
            start_dst(b, l + b)
        for b in range(NB):
            wait_dst(b)
            start_cscatter(b)
        return carry

    lax.fori_loop(1, CPT // NB, cgroup, 0)
    for b in range(NB):
        wait_cscatter(b)

    @pl.when(wid < EXTRA)
    def _():
        start_dst(0, NW * CPT + wid)
        wait_dst(0)
        start_cscatter(0)
        wait_cscatter(0)

    plsc.subcore_barrier()

    def cwriteback(r0, n):
        n0 = min(n, CW)
        pltpu.sync_copy(s_sh.at[pl.ds(r0, n0)], rows[1].at[pl.ds(0, n0)])
        pltpu.sync_copy(rows[1].at[pl.ds(0, n0)],
                        cnt_out.at[pl.ds(c * SR + r0, n0)])
        if n > CW:
            pltpu.sync_copy(s_sh.at[pl.ds(r0 + CW, n - CW)],
                            rows[2].at[pl.ds(0, n - CW)])
            pltpu.sync_copy(rows[2].at[pl.ds(0, n - CW)],
                            cnt_out.at[pl.ds(c * SR + r0 + CW, n - CW)])

    for k in range(NFULL):
        cwriteback(base + k * BW, BW)
    if TAIL:
        cwriteback(base + NFULL * BW, TAIL)


RB = 1000


def _tc_body(x_ref, s_ref, c_ref, wi_ref, wj_ref, b_ref, o_ref):
    cnt = c_ref[0, :, 0:1] + c_ref[1, :, 0:1]
    ssum = s_ref[0] + s_ref[1]
    smean = ssum / jnp.maximum(cnt, 1.0)
    o = (jnp.dot(x_ref[...], wi_ref[...], preferred_element_type=jnp.float32)
         + jnp.dot(smean, wj_ref[...], preferred_element_type=jnp.float32)
         + b_ref[...])
    o_ref[...] = jnp.where(cnt > 0.0, o, 0.0)


def kernel(x, edge_index, W, b):
    edge1d = edge_index.astype(jnp.int32).reshape(2 * N_EDGES)
    zrows = jnp.zeros((BW, D), jnp.float32)
    orows = jnp.ones((BW, D), jnp.float32)

    s_out, cnt_out = _sc_fused(x, edge1d, zrows, orows)
    s3 = s_out.reshape(NC, SR, D)
    c3 = cnt_out.reshape(NC, SR, D)

    wi_t = W[:, :D].T
    wj_t = W[:, D:].T
    b2 = b.reshape(1, D)

    out = pl.pallas_call(
        _tc_body,
        grid=(N_NODES // RB,),
        in_specs=[
            pl.BlockSpec((RB, D), lambda i: (i, 0)),
            pl.BlockSpec((NC, RB, D), lambda i: (0, i, 0)),
            pl.BlockSpec((NC, RB, D), lambda i: (0, i, 0)),
            pl.BlockSpec((D, D), lambda i: (0, 0)),
            pl.BlockSpec((D, D), lambda i: (0, 0)),
            pl.BlockSpec((1, D), lambda i: (0, 0)),
        ],
        out_specs=pl.BlockSpec((RB, D), lambda i: (i, 0)),
        out_shape=jax.ShapeDtypeStruct((N_NODES, D), jnp.float32),
    )(x, s3, c3, wi_t, wj_t, b2)
    return out


# speedup vs baseline: 13.6167x; 1.0088x over previous
import functools

import jax
import jax.numpy as jnp
from jax import lax
from jax.experimental import pallas as pl
from jax.experimental.pallas import tpu as pltpu
from jax.experimental.pallas import tpu_sc as plsc

N_NODES = 10000
N_EDGES = 320000
D = 128

NC, NS = 2, 16
NW = NC * NS
SR = 10112
STRIPE = SR // NS
BW = 128
NFULL = STRIPE // BW
TAIL = STRIPE - NFULL * BW

CW = 64
NCHUNKS = N_EDGES // CW
CPT = NCHUNKS // NW
EXTRA = NCHUNKS - CPT * NW
NB = 4

CWD = 128
NCHD = N_EDGES // CWD
CPTD = NCHD // NW
EXTRAD = NCHD - CPTD * NW
NBD = 8

_mesh = plsc.VectorSubcoreMesh(
    core_axis_name="c", subcore_axis_name="s", num_cores=NC, num_subcores=NS)


@functools.partial(
    pl.kernel,
    out_type=[
        jax.ShapeDtypeStruct((NC * SR, D), jnp.float32),
        jax.ShapeDtypeStruct((NC * SR, D), jnp.float32),
    ],
    mesh=_mesh,
    scratch_types=[
        [pltpu.VMEM((CW,), jnp.int32) for _ in range(NB)],
        [pltpu.VMEM((CW,), jnp.int32) for _ in range(NB)],
        [pltpu.VMEM((CW, D), jnp.float32) for _ in range(NB)],
        pltpu.VMEM_SHARED((SR, D), jnp.float32),
        [pltpu.SemaphoreType.DMA for _ in range(NB)],
        [pltpu.SemaphoreType.DMA for _ in range(NB)],
        [pltpu.SemaphoreType.DMA for _ in range(NB)],
        [pltpu.SemaphoreType.DMA for _ in range(NB)],
    ],
)
def _sc_fused(x_hbm, edge_hbm, zrows_hbm, orows_hbm, s_out, cnt_out,
              si, di, rows, s_sh, sg, ss, sis, sid):
    c = lax.axis_index("c")
    s = lax.axis_index("s")
    wid = c * NS + s

    pltpu.sync_copy(zrows_hbm.at[pl.ds(0, CW)], rows[0])
    pltpu.sync_copy(zrows_hbm.at[pl.ds(CW, CW)], rows[1])
    base = s * STRIPE

    def zero_block(r0, n):
        pltpu.sync_copy(rows[0].at[pl.ds(0, min(n, CW))],
                        s_sh.at[pl.ds(r0, min(n, CW))])
        if n > CW:
            pltpu.sync_copy(rows[1].at[pl.ds(0, n - CW)],
                            s_sh.at[pl.ds(r0 + CW, n - CW)])

    for k in range(NFULL):
        zero_block(base + k * BW, BW)
    if TAIL:
        zero_block(base + NFULL * BW, TAIL)
    plsc.subcore_barrier()

    cbase = wid * CPT

    def start_src(b, ch):
        off = pl.multiple_of(ch * CW, CW)
        pltpu.async_copy(edge_hbm.at[pl.ds(off, CW)], si[b], sis[b])

    def wait_src(b):
        pltpu.make_async_copy(edge_hbm.at[pl.ds(0, CW)], si[b], sis[b]).wait()

    def start_dst(b, ch):
        off = pl.multiple_of(ch * CW, CW)
        pltpu.async_copy(edge_hbm.at[pl.ds(N_EDGES + off, CW)], di[b], sid[b])

    def wait_dst(b):
        pltpu.make_async_copy(edge_hbm.at[pl.ds(0, CW)], di[b], sid[b]).wait()

    def start_gather(b):
        pltpu.async_copy(x_hbm.at[si[b]], rows[b], sg[b])

    def wait_gather(b):
        pltpu.make_async_copy(x_hbm.at[si[b]], rows[b], sg[b]).wait()

    def start_scatter(b):
        pltpu.async_copy(rows[b], s_sh.at[di[b]], ss[b], add=True)

    def wait_scatter(b):
        pltpu.make_async_copy(rows[b], s_sh.at[di[b]], ss[b]).wait()

    for b in range(NB):
        start_src(b, cbase + b)
        start_dst(b, cbase + b)
    for b in range(NB):
        wait_src(b)
        start_gather(b)
    for b in range(NB):
        wait_gather(b)
        wait_dst(b)
        start_scatter(b)

    def group(g, carry):
        l = cbase + g * NB
        for b in range(NB):
            wait_scatter(b)
            start_src(b, l + b)
            start_dst(b, l + b)
            wait_src(b)
            start_gather(b)
        for b in range(NB):
            wait_gather(b)
            wait_dst(b)
            start_scatter(b)
        return carry

    lax.fori_loop(1, CPT // NB, group, 0)
    for b in range(NB):
        wait_scatter(b)

    @pl.when(wid < EXTRA)
    def _():
        ch = NW * CPT + wid
        start_src(0, ch)
        start_dst(0, ch)
        wait_src(0)
        wait_dst(0)
        start_gather(0)
        wait_gather(0)
        start_scatter(0)
        wait_scatter(0)

    plsc.subcore_barrier()

    def writeback(r0, n):
        n0 = min(n, CW)
        pltpu.sync_copy(s_sh.at[pl.ds(r0, n0)], rows[0].at[pl.ds(0, n0)])
        pltpu.sync_copy(rows[0].at[pl.ds(0, n0)],
                        s_out.at[pl.ds(c * SR + r0, n0)])
        if n > CW:
            pltpu.sync_copy(s_sh.at[pl.ds(r0 + CW, n - CW)],
                            rows[1].at[pl.ds(0, n - CW)])
            pltpu.sync_copy(rows[1].at[pl.ds(0, n - CW)],
                            s_out.at[pl.ds(c * SR + r0 + CW, n - CW)])

    for k in range(NFULL):
        writeback(base + k * BW, BW)
    if TAIL:
        writeback(base + NFULL * BW, TAIL)

    pltpu.sync_copy(zrows_hbm.at[pl.ds(0, CW)], rows[0])
    pltpu.sync_copy(zrows_hbm.at[pl.ds(CW, CW)], rows[1])
    for k in range(NFULL):
        zero_block(base + k * BW, BW)
    if TAIL:
        zero_block(base + NFULL * BW, TAIL)
    pltpu.sync_copy(orows_hbm.at[pl.ds(0, CW)], rows[0])
    plsc.subcore_barrier()

    def start_cscatter(b):
        pltpu.async_copy(rows[0], s_sh.at[di[b]], ss[b], add=True)

    def wait_cscatter(b):
        pltpu.make_async_copy(rows[0], s_sh.at[di[b]], ss[b]).wait()

    for b in range(NB):
        start_dst(b, cbase + b)
    for b in range(NB):
        wait_dst(b)
        start_cscatter(b)

    def cgroup(g, carry):
        l = cbase + g * NB
        for b in range(NB):
            wait_cscatter(b)
            start_dst(b, l + b)
        for b in range(NB):
            wait_dst(b)
            start_cscatter(b)
        return carry

    lax.fori_loop(1, CPT // NB, cgroup, 0)
    for b in range(NB):
        wait_cscatter(b)

    @pl.when(wid < EXTRA)
    def _():
        start_dst(0, NW * CPT + wid)
        wait_dst(0)
        start_cscatter(0)
        wait_cscatter(0)

    plsc.subcore_barrier()

    def cwriteback(r0, n):
        n0 = min(n, CW)
        pltpu.sync_copy(s_sh.at[pl.ds(r0, n0)], rows[1].at[pl.ds(0, n0)])
        pltpu.sync_copy(rows[1].at[pl.ds(0, n0)],
                        cnt_out.at[pl.ds(c * SR + r0, n0)])
        if n > CW:
            pltpu.sync_copy(s_sh.at[pl.ds(r0 + CW, n - CW)],
                            rows[2].at[pl.ds(0, n - CW)])
            pltpu.sync_copy(rows[2].at[pl.ds(0, n - CW)],
                            cnt_out.at[pl.ds(c * SR + r0 + CW, n - CW)])

    for k in range(NFULL):
        cwriteback(base + k * BW, BW)
    if TAIL:
        cwriteback(base + NFULL * BW, TAIL)


RB = 2000


def _tc_body(x_ref, s_ref, c_ref, wi_ref, wj_ref, b_ref, o_ref):
    cnt = c_ref[0, :, 0:1] + c_ref[1, :, 0:1]
    ssum = s_ref[0] + s_ref[1]
    smean = ssum / jnp.maximum(cnt, 1.0)
    o = (jnp.dot(x_ref[...], wi_ref[...], preferred_element_type=jnp.float32)
         + jnp.dot(smean, wj_ref[...], preferred_element_type=jnp.float32)
         + b_ref[...])
    o_ref[...] = jnp.where(cnt > 0.0, o, 0.0)


def kernel(x, edge_index, W, b):
    edge1d = edge_index.astype(jnp.int32).reshape(2 * N_EDGES)
    zrows = jnp.zeros((BW, D), jnp.float32)
    orows = jnp.ones((BW, D), jnp.float32)

    s_out, cnt_out = _sc_fused(x, edge1d, zrows, orows)
    s3 = s_out.reshape(NC, SR, D)
    c3 = cnt_out.reshape(NC, SR, D)

    wi_t = W[:, :D].T
    wj_t = W[:, D:].T
    b2 = b.reshape(1, D)

    out = pl.pallas_call(
        _tc_body,
        grid=(N_NODES // RB,),
        in_specs=[
            pl.BlockSpec((RB, D), lambda i: (i, 0)),
            pl.BlockSpec((NC, RB, D), lambda i: (0, i, 0)),
            pl.BlockSpec((NC, RB, D), lambda i: (0, i, 0)),
            pl.BlockSpec((D, D), lambda i: (0, 0)),
            pl.BlockSpec((D, D), lambda i: (0, 0)),
            pl.BlockSpec((1, D), lambda i: (0, 0)),
        ],
        out_specs=pl.BlockSpec((RB, D), lambda i: (i, 0)),
        out_shape=jax.ShapeDtypeStruct((N_NODES, D), jnp.float32),
    )(x, s3, c3, wi_t, wj_t, b2)
    return out


# speedup vs baseline: 13.6333x; 1.0012x over previous
import functools

import jax
import jax.numpy as jnp
from jax import lax
from jax.experimental import pallas as pl
from jax.experimental.pallas import tpu as pltpu
from jax.experimental.pallas import tpu_sc as plsc

N_NODES = 10000
N_EDGES = 320000
D = 128

NC, NS = 2, 16
NW = NC * NS
SR = 10112
STRIPE = SR // NS
BW = 128
NFULL = STRIPE // BW
TAIL = STRIPE - NFULL * BW

CW = 64
NCHUNKS = N_EDGES // CW
CPT = NCHUNKS // NW
EXTRA = NCHUNKS - CPT * NW
NB = 4


_mesh = plsc.VectorSubcoreMesh(
    core_axis_name="c", subcore_axis_name="s", num_cores=NC, num_subcores=NS)


@functools.partial(
    pl.kernel,
    out_type=[
        jax.ShapeDtypeStruct((NC * SR, D), jnp.float32),
        jax.ShapeDtypeStruct((NC * SR, D), jnp.float32),
    ],
    mesh=_mesh,
    scratch_types=[
        [pltpu.VMEM((CW,), jnp.int32) for _ in range(NB)],
        [pltpu.VMEM((CW,), jnp.int32) for _ in range(NB)],
        [pltpu.VMEM((CW, D), jnp.float32) for _ in range(NB)],
        pltpu.VMEM_SHARED((SR, D), jnp.float32),
        [pltpu.SemaphoreType.DMA for _ in range(NB)],
        [pltpu.SemaphoreType.DMA for _ in range(NB)],
        [pltpu.SemaphoreType.DMA for _ in range(NB)],
        [pltpu.SemaphoreType.DMA for _ in range(NB)],
    ],
)
def _sc_fused(x_hbm, edge_hbm, zrows_hbm, orows_hbm, s_out, cnt_out,
              si, di, rows, s_sh, sg, ss, sis, sid):
    c = lax.axis_index("c")
    s = lax.axis_index("s")
    wid = c * NS + s

    pltpu.sync_copy(zrows_hbm.at[pl.ds(0, CW)], rows[0])
    pltpu.sync_copy(zrows_hbm.at[pl.ds(CW, CW)], rows[1])
    base = s * STRIPE

    def zero_block(r0, n):
        pltpu.sync_copy(rows[0].at[pl.ds(0, min(n, CW))],
                        s_sh.at[pl.ds(r0, min(n, CW))])
        if n > CW:
            pltpu.sync_copy(rows[1].at[pl.ds(0, n - CW)],
                            s_sh.at[pl.ds(r0 + CW, n - CW)])

    for k in range(NFULL):
        zero_block(base + k * BW, BW)
    if TAIL:
        zero_block(base + NFULL * BW, TAIL)
    plsc.subcore_barrier()

    cbase = wid * CPT

    def start_src(b, ch):
        off = pl.multiple_of(ch * CW, CW)
        pltpu.async_copy(edge_hbm.at[pl.ds(off, CW)], si[b], sis[b])

    def wait_src(b):
        pltpu.make_async_copy(edge_hbm.at[pl.ds(0, CW)], si[b], sis[b]).wait()

    def start_dst(b, ch):
        off = pl.multiple_of(ch * CW, CW)
        pltpu.async_copy(edge_hbm.at[pl.ds(N_EDGES + off, CW)], di[b], sid[b])

    def wait_dst(b):
        pltpu.make_async_copy(edge_hbm.at[pl.ds(0, CW)], di[b], sid[b]).wait()

    def start_gather(b):
        pltpu.async_copy(x_hbm.at[si[b]], rows[b], sg[b])

    def wait_gather(b):
        pltpu.make_async_copy(x_hbm.at[si[b]], rows[b], sg[b]).wait()

    def start_scatter(b):
        pltpu.async_copy(rows[b], s_sh.at[di[b]], ss[b], add=True)

    def wait_scatter(b):
        pltpu.make_async_copy(rows[b], s_sh.at[di[b]], ss[b]).wait()

    for b in range(NB):
        start_src(b, cbase + b)
        start_dst(b, cbase + b)
    for b in range(NB):
        wait_src(b)
        start_gather(b)
    for b in range(NB):
        wait_gather(b)
        wait_dst(b)
        start_scatter(b)

    def group(g, carry):
        l = cbase + g * NB
        for b in range(NB):
            wait_scatter(b)
            start_src(b, l + b)
            start_dst(b, l + b)
            wait_src(b)
            start_gather(b)
        for b in range(NB):
            wait_gather(b)
            wait_dst(b)
            start_scatter(b)
        return carry

    lax.fori_loop(1, CPT // NB, group, 0)
    for b in range(NB):
        wait_scatter(b)

    @pl.when(wid < EXTRA)
    def _():
        ch = NW * CPT + wid
        start_src(0, ch)
        start_dst(0, ch)
        wait_src(0)
        wait_dst(0)
        start_gather(0)
        wait_gather(0)
        start_scatter(0)
        wait_scatter(0)

    plsc.subcore_barrier()

    def writeback(r0, n):
        n0 = min(n, CW)
        pltpu.sync_copy(s_sh.at[pl.ds(r0, n0)], rows[0].at[pl.ds(0, n0)])
        pltpu.sync_copy(rows[0].at[pl.ds(0, n0)],
                        s_out.at[pl.ds(c * SR + r0, n0)])
        if n > CW:
            pltpu.sync_copy(s_sh.at[pl.ds(r0 + CW, n - CW)],
                            rows[1].at[pl.ds(0, n - CW)])
            pltpu.sync_copy(rows[1].at[pl.ds(0, n - CW)],
                            s_out.at[pl.ds(c * SR + r0 + CW, n - CW)])

    for k in range(NFULL):
        writeback(base + k * BW, BW)
    if TAIL:
        writeback(base + NFULL * BW, TAIL)

    pltpu.sync_copy(zrows_hbm.at[pl.ds(0, CW)], rows[0])
    pltpu.sync_copy(zrows_hbm.at[pl.ds(CW, CW)], rows[1])
    for k in range(NFULL):
        zero_block(base + k * BW, BW)
    if TAIL:
        zero_block(base + NFULL * BW, TAIL)
    pltpu.sync_copy(orows_hbm.at[pl.ds(0, CW)], rows[0])
    plsc.subcore_barrier()

    def start_cscatter(b):
        pltpu.async_copy(rows[0], s_sh.at[di[b]], ss[b], add=True)

    def wait_cscatter(b):
        pltpu.make_async_copy(rows[0], s_sh.at[di[b]], ss[b]).wait()

    for b in range(NB):
        start_dst(b, cbase + b)
    for b in range(NB):
        wait_dst(b)
        start_cscatter(b)

    def cgroup(g, carry):
        l = cbase + g * NB
        for b in range(NB):
            wait_cscatter(b)
            start_dst(b, l + b)
        for b in range(NB):
            wait_dst(b)
            start_cscatter(b)
        return carry

    lax.fori_loop(1, CPT // NB, cgroup, 0)
    for b in range(NB):
        wait_cscatter(b)

    @pl.when(wid < EXTRA)
    def _():
        start_dst(0, NW * CPT + wid)
        wait_dst(0)
        start_cscatter(0)
        wait_cscatter(0)

    plsc.subcore_barrier()

    def cwriteback(r0, n):
        n0 = min(n, CW)
        pltpu.sync_copy(s_sh.at[pl.ds(r0, n0)], rows[1].at[pl.ds(0, n0)])
        pltpu.sync_copy(rows[1].at[pl.ds(0, n0)],
                        cnt_out.at[pl.ds(c * SR + r0, n0)])
        if n > CW:
            pltpu.sync_copy(s_sh.at[pl.ds(r0 + CW, n - CW)],
                            rows[2].at[pl.ds(0, n - CW)])
            pltpu.sync_copy(rows[2].at[pl.ds(0, n - CW)],
                            cnt_out.at[pl.ds(c * SR + r0 + CW, n - CW)])

    for k in range(NFULL):
        cwriteback(base + k * BW, BW)
    if TAIL:
        cwriteback(base + NFULL * BW, TAIL)


RB = 2000


def _tc_body(x_ref, s_ref, c_ref, wi_ref, wj_ref, b_ref, o_ref):
    cnt = c_ref[0, :, 0:1] + c_ref[1, :, 0:1]
    ssum = s_ref[0] + s_ref[1]
    smean = ssum / jnp.maximum(cnt, 1.0)
    o = (jnp.dot(x_ref[...], wi_ref[...], preferred_element_type=jnp.float32)
         + jnp.dot(smean, wj_ref[...], preferred_element_type=jnp.float32)
         + b_ref[...])
    o_ref[...] = jnp.where(cnt > 0.0, o, 0.0)


def kernel(x, edge_index, W, b):
    edge1d = edge_index.astype(jnp.int32).reshape(2 * N_EDGES)
    zrows = jnp.zeros((BW, D), jnp.float32)
    orows = jnp.ones((BW, D), jnp.float32)

    s_out, cnt_out = _sc_fused(x, edge1d, zrows, orows)
    s3 = s_out.reshape(NC, SR, D)
    c3 = cnt_out.reshape(NC, SR, D)

    wi_t = W[:, :D].T
    wj_t = W[:, D:].T
    b2 = b.reshape(1, D)

    out = pl.pallas_call(
        _tc_body,
        grid=(N_NODES // RB,),
        in_specs=[
            pl.BlockSpec((RB, D), lambda i: (i, 0)),
            pl.BlockSpec((NC, RB, D), lambda i: (0, i, 0)),
            pl.BlockSpec((NC, RB, D), lambda i: (0, i, 0)),
            pl.BlockSpec((D, D), lambda i: (0, 0)),
            pl.BlockSpec((D, D), lambda i: (0, 0)),
            pl.BlockSpec((1, D), lambda i: (0, 0)),
        ],
        out_specs=pl.BlockSpec((RB, D), lambda i: (i, 0)),
        out_shape=jax.ShapeDtypeStruct((N_NODES, D), jnp.float32),
    )(x, s3, c3, wi_t, wj_t, b2)
    return out
